# baseline (speedup 1.0000x reference)
import functools

import jax
import jax.numpy as jnp
from jax import lax
from jax.experimental import pallas as pl
from jax.experimental.pallas import tpu as pltpu
from jax.experimental.pallas import tpu_sc as plsc

B = 16384
D = 64
NCOL = 10
NC, NS = 2, 16
NW = NC * NS
CH = 512
NCHUNK = (B * NCOL) // (NW * CH)
BI = B // NW
BJ = 8 * B // NW

_MESH = plsc.VectorSubcoreMesh(core_axis_name="c", subcore_axis_name="s")


@functools.partial(
    pl.kernel,
    mesh=_MESH,
    compiler_params=pltpu.CompilerParams(use_tc_tiling_on_sc=False),
    out_type=[
        jax.ShapeDtypeStruct((B, D), jnp.float32),
        jax.ShapeDtypeStruct((B, D), jnp.float32),
        jax.ShapeDtypeStruct((8 * B, D), jnp.float32),
    ],
    scratch_types=(
        [pltpu.VMEM((CH,), jnp.int32) for _ in range(NCHUNK)]
        + [
            pltpu.VMEM((CH, D), jnp.float32),
            pltpu.VMEM((CH, D), jnp.float32),
            pltpu.SemaphoreType.DMA,
            pltpu.SemaphoreType.DMA,
        ]
    ),
)
def _gather(table_hbm, idx_hbm, o_i, o_k, o_j, *scratch):
    idxv = scratch[:NCHUNK]
    buf0, buf1, sem0, sem1 = scratch[NCHUNK:]
    wid = lax.axis_index("s") * NC + lax.axis_index("c")
    for t in range(NCHUNK):
        pltpu.sync_copy(idx_hbm.at[wid, t], idxv[t])

    def out_slice(t):
        if t == 0:
            return o_i.at[pl.ds(wid * BI, CH)]
        if t == 1:
            return o_k.at[pl.ds(wid * BI, CH)]
        return o_j.at[pl.ds(wid * BJ + (t - 2) * CH, CH)]

    bufs = (buf0, buf1)
    sems = (sem0, sem1)
    copies = [None, None]
    for t in range(NCHUNK):
        b = t % 2
        copies[b] = pltpu.async_copy(table_hbm.at[idxv[t]], bufs[b], sems[b])
        if t >= 1:
            copies[1 - b].wait()
            pltpu.sync_copy(bufs[1 - b], out_slice(t - 1))
    last = (NCHUNK - 1) % 2
    copies[last].wait()
    pltpu.sync_copy(bufs[last], out_slice(NCHUNK - 1))


def kernel(items, table):
    items = items.astype(jnp.int32)
    idx_all = jnp.concatenate(
        [
            items[:, 0].reshape(NW, 1, CH),
            items[:, 1].reshape(NW, 1, CH),
            items[:, 2:].reshape(NW, 8, CH),
        ],
        axis=1,
    )
    o_i, o_k, o_j = _gather(table, idx_all)
    return (o_i, o_k, o_j.reshape(B, 8, D))


# speedup vs baseline: 1.0638x; 1.0638x over previous
import functools

import jax
import jax.numpy as jnp
from jax import lax
from jax.experimental import pallas as pl
from jax.experimental.pallas import tpu as pltpu
from jax.experimental.pallas import tpu_sc as plsc

B = 16384
D = 64
DP = 128
NCOL = 10
NC, NS = 2, 16
NW = NC * NS
CH = 256
NCHUNK = (B * NCOL) // (NW * CH)
BI = B // NW
BJ = 8 * B // NW
CI = BI // CH
CJ = BJ // CH

_MESH = plsc.VectorSubcoreMesh(core_axis_name="c", subcore_axis_name="s")


@functools.partial(
    pl.kernel,
    mesh=_MESH,
    compiler_params=pltpu.CompilerParams(use_tc_tiling_on_sc=False),
    out_type=[
        jax.ShapeDtypeStruct((B, DP), jnp.float32),
        jax.ShapeDtypeStruct((B, DP), jnp.float32),
        jax.ShapeDtypeStruct((8 * B, DP), jnp.float32),
    ],
    scratch_types=(
        [pltpu.VMEM((CH,), jnp.int32) for _ in range(NCHUNK)]
        + [
            pltpu.VMEM((CH, DP), jnp.float32),
            pltpu.VMEM((CH, DP), jnp.float32),
            pltpu.SemaphoreType.DMA,
            pltpu.SemaphoreType.DMA,
        ]
    ),
)
def _gather(table_hbm, idx_hbm, o_i, o_k, o_j, *scratch):
    idxv = scratch[:NCHUNK]
    buf0, buf1, sem0, sem1 = scratch[NCHUNK:]
    wid = lax.axis_index("s") * NC + lax.axis_index("c")
    for t in range(NCHUNK):
        pltpu.sync_copy(idx_hbm.at[wid, t], idxv[t])

    def out_slice(t):
        if t < CI:
            return o_i.at[pl.ds(wid * BI + t * CH, CH)]
        if t < 2 * CI:
            return o_k.at[pl.ds(wid * BI + (t - CI) * CH, CH)]
        return o_j.at[pl.ds(wid * BJ + (t - 2 * CI) * CH, CH)]

    bufs = (buf0, buf1)
    sems = (sem0, sem1)
    copies = [None, None]
    for t in range(NCHUNK):
        b = t % 2
        copies[b] = pltpu.async_copy(table_hbm.at[idxv[t]], bufs[b], sems[b])
        if t >= 1:
            copies[1 - b].wait()
            pltpu.sync_copy(bufs[1 - b], out_slice(t - 1))
    last = (NCHUNK - 1) % 2
    copies[last].wait()
    pltpu.sync_copy(bufs[last], out_slice(NCHUNK - 1))


def kernel(items, table):
    items = items.astype(jnp.int32)
    table_p = jnp.pad(table, ((0, 0), (0, DP - D)))
    idx_all = jnp.concatenate(
        [
            items[:, 0].reshape(NW, CI, CH),
            items[:, 1].reshape(NW, CI, CH),
            items[:, 2:].reshape(NW, CJ, CH),
        ],
        axis=1,
    )
    o_i, o_k, o_j = _gather(table_p, idx_all)
    return (o_i[:, :D], o_k[:, :D], o_j[:, :D].reshape(B, 8, D))
